# baseline (speedup 1.0000x reference)
import jax
import jax.numpy as jnp
from jax import lax
from jax.experimental import pallas as pl
from jax.experimental.pallas import tpu as pltpu
from jax.experimental.pallas import tpu_sc as plsc

SEQ_LEN = 8192
MODEL_DIM = 1024

_info = plsc.get_sparse_core_info()
_NC, _NS = _info.num_cores, _info.num_subcores
_NW = _NC * _NS
_ROWS_PER_W = SEQ_LEN // _NW
_CHUNK = 32
_NCHUNKS = _ROWS_PER_W // _CHUNK
_NBUF = 3


def _copy_body(table_hbm, out_hbm, buf0, buf1, buf2, sl0, sl1, sl2, ss0, ss1, ss2):
    wid = lax.axis_index("s") * _NC + lax.axis_index("c")
    base = wid * _ROWS_PER_W
    bufs = (buf0, buf1, buf2)
    sem_l = (sl0, sl1, sl2)
    sem_s = (ss0, ss1, ss2)

    def load(i):
        b = i % _NBUF
        r0 = base + i * _CHUNK
        return pltpu.make_async_copy(
            table_hbm.at[pl.ds(r0, _CHUNK), :], bufs[b], sem_l[b])

    def store(i):
        b = i % _NBUF
        r0 = base + i * _CHUNK
        return pltpu.make_async_copy(
            bufs[b], out_hbm.at[pl.ds(r0, _CHUNK), :], sem_s[b])

    for i in range(_NBUF):
        load(i).start()
    for i in range(_NCHUNKS):
        load(i).wait()
        store(i).start()
        p = i - 1
        if p >= 0 and p + _NBUF < _NCHUNKS:
            store(p).wait()
            load(p + _NBUF).start()
    for i in range(_NCHUNKS - _NBUF, _NCHUNKS):
        store(i).wait()


def kernel(x, emb_weight):
    mesh = plsc.VectorSubcoreMesh(core_axis_name="c", subcore_axis_name="s")
    copy = pl.kernel(
        _copy_body,
        mesh=mesh,
        out_type=jax.ShapeDtypeStruct((SEQ_LEN, MODEL_DIM), jnp.float32),
        scratch_types=[
            pltpu.VMEM((_CHUNK, MODEL_DIM), jnp.float32),
            pltpu.VMEM((_CHUNK, MODEL_DIM), jnp.float32),
            pltpu.VMEM((_CHUNK, MODEL_DIM), jnp.float32),
            pltpu.SemaphoreType.DMA,
            pltpu.SemaphoreType.DMA,
            pltpu.SemaphoreType.DMA,
            pltpu.SemaphoreType.DMA,
            pltpu.SemaphoreType.DMA,
            pltpu.SemaphoreType.DMA,
        ],
    )
    return copy(emb_weight)
